# Initial kernel scaffold; baseline (speedup 1.0000x reference)
import functools

import jax
import jax.numpy as jnp
from jax import lax
from jax.experimental import pallas as pl
from jax.experimental.pallas import tpu as pltpu
from jax.experimental.pallas import tpu_sc as plsc

N = 10000
E = 320000
H = 128
H2 = 64
EPS = 1e-07


_BN = 2000
_BE = 4000


def _enc_node_body(x_ref, w_ref, b_ref, out_ref):
    h = jnp.dot(x_ref[...], w_ref[...], preferred_element_type=jnp.float32)
    h = h + b_ref[...]
    out_ref[0] = h[:, :H2]
    out_ref[1] = h[:, H2:]


def _enc_node(x, W, b):
    return pl.pallas_call(
        _enc_node_body,
        grid=(N // _BN,),
        in_specs=[
            pl.BlockSpec((_BN, 3), lambda i: (i, 0)),
            pl.BlockSpec((3, H), lambda i: (0, 0)),
            pl.BlockSpec((1, H), lambda i: (0, 0)),
        ],
        out_specs=pl.BlockSpec((2, _BN, H2), lambda i: (0, i, 0)),
        out_shape=jax.ShapeDtypeStruct((2, N, H2), jnp.float32),
    )(x, W, b)


def _enc_edge_body(a_ref, w_ref, b_ref, out_ref):
    h = jnp.dot(a_ref[...], w_ref[...], preferred_element_type=jnp.float32)
    h = h + b_ref[...]
    out_ref[0] = h[:, :H2]
    out_ref[1] = h[:, H2:]


def _enc_edge(attr, W, b):
    return pl.pallas_call(
        _enc_edge_body,
        grid=(E // _BE,),
        in_specs=[
            pl.BlockSpec((_BE, 7), lambda i: (i, 0)),
            pl.BlockSpec((7, H), lambda i: (0, 0)),
            pl.BlockSpec((1, H), lambda i: (0, 0)),
        ],
        out_specs=pl.BlockSpec((2, _BE, H2), lambda i: (0, i, 0)),
        out_shape=jax.ShapeDtypeStruct((2, E, H2), jnp.float32),
    )(attr, W, b)


def _mlp_body(relu_out, h_ref, a_ref, w_ref, b_ref, g_ref, be_ref, out_ref):
    hp = jnp.concatenate([h_ref[0] + a_ref[0], h_ref[1] + a_ref[1]], axis=-1)
    t = jnp.dot(hp, w_ref[...], preferred_element_type=jnp.float32)
    t = t + b_ref[...]
    mu = jnp.mean(t, axis=-1, keepdims=True)
    var = jnp.mean((t - mu) * (t - mu), axis=-1, keepdims=True)
    y = (t - mu) / jnp.sqrt(var + 1e-5) * g_ref[...] + be_ref[...]
    if relu_out:
        y = jnp.maximum(y, 0.0)
        out_ref[0] = y[:, :H2]
        out_ref[1] = y[:, H2:]
    else:
        out_ref[...] = y


def _mlp(hs, aggr, Wc, bc, g, be, relu_out):
    if relu_out:
        out_spec = pl.BlockSpec((2, _BN, H2), lambda i: (0, i, 0))
        out_shape = jax.ShapeDtypeStruct((2, N, H2), jnp.float32)
    else:
        out_spec = pl.BlockSpec((_BN, H), lambda i: (i, 0))
        out_shape = jax.ShapeDtypeStruct((N, H), jnp.float32)
    return pl.pallas_call(
        functools.partial(_mlp_body, relu_out),
        grid=(N // _BN,),
        in_specs=[
            pl.BlockSpec((2, _BN, H2), lambda i: (0, i, 0)),
            pl.BlockSpec((2, _BN, H2), lambda i: (0, i, 0)),
            pl.BlockSpec((H, H), lambda i: (0, 0)),
            pl.BlockSpec((1, H), lambda i: (0, 0)),
            pl.BlockSpec((1, H), lambda i: (0, 0)),
            pl.BlockSpec((1, H), lambda i: (0, 0)),
        ],
        out_specs=out_spec,
        out_shape=out_shape,
    )(hs, aggr, Wc, bc, g, be)


_NSUB = 16
_C = 128
_EPT = E // _NSUB
_NFULL = _EPT // _C
_CT = _EPT - _NFULL * _C
_NPT = N // _NSUB
_FC = 125

_mesh = plsc.VectorSubcoreMesh(core_axis_name="c", subcore_axis_name="s")


def _conv_body(h_hbm, emb_hbm, src_hbm, dst_hbm, out_hbm,
               srcv, dstv, srcvt, dstvt, hrows, erows, ebuf, wbuf,
               S_sh, W_sh, sem):
    cid = lax.axis_index("c")
    sid = lax.axis_index("s")

    zero = jnp.zeros((16,), jnp.float32)

    def zbody(e, carry):
        for k in range(4):
            ebuf[e, pl.ds(k * 16, 16)] = zero
        return carry

    lax.fori_loop(0, _FC, zbody, 0, unroll=False)
    for j in range(N // (_FC * _NSUB)):
        nb = sid * _NPT + j * _FC
        pltpu.sync_copy(ebuf.at[pl.ds(0, _FC)], S_sh.at[pl.ds(nb, _FC)])
        pltpu.sync_copy(ebuf.at[pl.ds(0, _FC)], W_sh.at[pl.ds(nb, _FC)])
    plsc.subcore_barrier()

    def process(off, csz, sv, dv):
        pltpu.sync_copy(src_hbm.at[pl.ds(off, csz)], sv)
        pltpu.sync_copy(dst_hbm.at[pl.ds(off, csz)], dv)
        pltpu.async_copy(h_hbm.at[cid, sv], hrows.at[pl.ds(0, csz)], sem).wait()
        pltpu.sync_copy(emb_hbm.at[cid, pl.ds(off, csz)], erows.at[pl.ds(0, csz)])

        def cbody(e, carry):
            for k in range(4):
                sl = pl.ds(k * 16, 16)
                msg = jnp.maximum(hrows[e, sl] + erows[e, sl], 0.0) + EPS
                ex = jnp.exp(msg)
                ebuf[e, sl] = ex
                wbuf[e, sl] = ex * msg
            return carry

        lax.fori_loop(0, csz, cbody, 0, unroll=False)
        pltpu.sync_copy(ebuf.at[pl.ds(0, csz)], S_sh.at[dv], add=True)
        pltpu.sync_copy(wbuf.at[pl.ds(0, csz)], W_sh.at[dv], add=True)

    base = sid * _EPT

    def chunk(i, carry):
        process(base + i * _C, _C, srcv, dstv)
        return carry

    lax.fori_loop(0, _NFULL, chunk, 0, unroll=False)
    if _CT:
        process(base + _NFULL * _C, _CT, srcvt, dstvt)

    plsc.subcore_barrier()

    for j in range(N // (_FC * _NSUB)):
        nb = sid * _NPT + j * _FC
        pltpu.sync_copy(S_sh.at[pl.ds(nb, _FC)], ebuf.at[pl.ds(0, _FC)])
        pltpu.sync_copy(W_sh.at[pl.ds(nb, _FC)], wbuf.at[pl.ds(0, _FC)])

        def fbody(e, carry):
            for k in range(4):
                sl = pl.ds(k * 16, 16)
                hrows[e, sl] = wbuf[e, sl] / (ebuf[e, sl] + 1e-16)
            return carry

        lax.fori_loop(0, _FC, fbody, 0, unroll=False)
        pltpu.sync_copy(hrows.at[pl.ds(0, _FC)], out_hbm.at[cid, pl.ds(nb, _FC)])


def _conv_sc(h_split, emb_split, src, dst):
    kern = pl.kernel(
        _conv_body,
        out_type=jax.ShapeDtypeStruct((2, N, H2), jnp.float32),
        mesh=_mesh,
        scratch_types=[
            pltpu.VMEM((_C,), jnp.int32),
            pltpu.VMEM((_C,), jnp.int32),
            pltpu.VMEM((_CT,), jnp.int32),
            pltpu.VMEM((_CT,), jnp.int32),
            pltpu.VMEM((_C, H2), jnp.float32),
            pltpu.VMEM((_C, H2), jnp.float32),
            pltpu.VMEM((_C, H2), jnp.float32),
            pltpu.VMEM((_C, H2), jnp.float32),
            pltpu.VMEM_SHARED((N, H2), jnp.float32),
            pltpu.VMEM_SHARED((N, H2), jnp.float32),
            pltpu.SemaphoreType.DMA,
        ],
    )
    return kern(h_split, emb_split, src, dst)


def kernel(x, edge_index, edge_attr, W_node, b_node, W_edge, b_edge,
           Wc0, bc0, Wc1, bc1, g0, be0, g1, be1):
    src = edge_index[0]
    dst = edge_index[1]
    b_node = b_node.reshape(1, H)
    b_edge = b_edge.reshape(1, H)
    bc0 = bc0.reshape(1, H)
    bc1 = bc1.reshape(1, H)
    g0 = g0.reshape(1, H)
    g1 = g1.reshape(1, H)
    be0 = be0.reshape(1, H)
    be1 = be1.reshape(1, H)

    h0 = _enc_node(x, W_node, b_node)
    emb = _enc_edge(edge_attr, W_edge, b_edge)
    a1 = _conv_sc(h0, emb, src, dst)
    h2 = _mlp(h0, a1, Wc0, bc0, g0, be0, relu_out=True)
    a2 = _conv_sc(h2, emb, src, dst)
    return _mlp(h2, a2, Wc1, bc1, g1, be1, relu_out=False)


# speedup vs baseline: 4.9166x; 4.9166x over previous
import functools

import jax
import jax.numpy as jnp
from jax import lax
from jax.experimental import pallas as pl
from jax.experimental.pallas import tpu as pltpu
from jax.experimental.pallas import tpu_sc as plsc

N = 10000
E = 320000
H = 128
H2 = 64
EPS = 1e-07


_BN = 2000
_BE = 4000


def _enc_node_body(x_ref, w_ref, b_ref, out_ref):
    h = jnp.dot(x_ref[...], w_ref[...], preferred_element_type=jnp.float32)
    h = h + b_ref[...]
    out_ref[0] = h[:, :H2]
    out_ref[1] = h[:, H2:]


def _enc_node(x, W, b):
    return pl.pallas_call(
        _enc_node_body,
        grid=(N // _BN,),
        in_specs=[
            pl.BlockSpec((_BN, 3), lambda i: (i, 0)),
            pl.BlockSpec((3, H), lambda i: (0, 0)),
            pl.BlockSpec((1, H), lambda i: (0, 0)),
        ],
        out_specs=pl.BlockSpec((2, _BN, H2), lambda i: (0, i, 0)),
        out_shape=jax.ShapeDtypeStruct((2, N, H2), jnp.float32),
    )(x, W, b)


def _enc_edge_body(a_ref, w_ref, b_ref, out_ref):
    h = jnp.dot(a_ref[...], w_ref[...], preferred_element_type=jnp.float32)
    h = h + b_ref[...]
    out_ref[0] = h[:, :H2]
    out_ref[1] = h[:, H2:]


def _enc_edge(attr, W, b):
    return pl.pallas_call(
        _enc_edge_body,
        grid=(E // _BE,),
        in_specs=[
            pl.BlockSpec((_BE, 7), lambda i: (i, 0)),
            pl.BlockSpec((7, H), lambda i: (0, 0)),
            pl.BlockSpec((1, H), lambda i: (0, 0)),
        ],
        out_specs=pl.BlockSpec((2, _BE, H2), lambda i: (0, i, 0)),
        out_shape=jax.ShapeDtypeStruct((2, E, H2), jnp.float32),
    )(attr, W, b)


def _mlp_body(relu_out, h_ref, a_ref, w_ref, b_ref, g_ref, be_ref, out_ref):
    hp = jnp.concatenate([h_ref[0] + a_ref[0], h_ref[1] + a_ref[1]], axis=-1)
    t = jnp.dot(hp, w_ref[...], preferred_element_type=jnp.float32)
    t = t + b_ref[...]
    mu = jnp.mean(t, axis=-1, keepdims=True)
    var = jnp.mean((t - mu) * (t - mu), axis=-1, keepdims=True)
    y = (t - mu) / jnp.sqrt(var + 1e-5) * g_ref[...] + be_ref[...]
    if relu_out:
        y = jnp.maximum(y, 0.0)
        out_ref[0] = y[:, :H2]
        out_ref[1] = y[:, H2:]
    else:
        out_ref[...] = y


def _mlp(hs, aggr, Wc, bc, g, be, relu_out):
    if relu_out:
        out_spec = pl.BlockSpec((2, _BN, H2), lambda i: (0, i, 0))
        out_shape = jax.ShapeDtypeStruct((2, N, H2), jnp.float32)
    else:
        out_spec = pl.BlockSpec((_BN, H), lambda i: (i, 0))
        out_shape = jax.ShapeDtypeStruct((N, H), jnp.float32)
    return pl.pallas_call(
        functools.partial(_mlp_body, relu_out),
        grid=(N // _BN,),
        in_specs=[
            pl.BlockSpec((2, _BN, H2), lambda i: (0, i, 0)),
            pl.BlockSpec((2, _BN, H2), lambda i: (0, i, 0)),
            pl.BlockSpec((H, H), lambda i: (0, 0)),
            pl.BlockSpec((1, H), lambda i: (0, 0)),
            pl.BlockSpec((1, H), lambda i: (0, 0)),
            pl.BlockSpec((1, H), lambda i: (0, 0)),
        ],
        out_specs=out_spec,
        out_shape=out_shape,
    )(hs, aggr, Wc, bc, g, be)


_NSUB = 16
_C = 128
_EPT = E // _NSUB
_NFULL = _EPT // _C
_CT = _EPT - _NFULL * _C
_NPT = N // _NSUB
_FC = 125

_mesh = plsc.VectorSubcoreMesh(core_axis_name="c", subcore_axis_name="s")


def _conv_body(h_hbm, emb_hbm, src_hbm, dst_hbm, out_hbm,
               srcv, dstv, srcvt, dstvt, hrows, erows, ebuf, wbuf,
               S_sh, W_sh, sem):
    cid = lax.axis_index("c")
    sid = lax.axis_index("s")

    zero = jnp.zeros((16,), jnp.float32)

    def zbody(e, carry):
        for k in range(4):
            ebuf[e, pl.ds(k * 16, 16)] = zero
        return carry

    lax.fori_loop(0, _FC, zbody, 0, unroll=False)
    for j in range(N // (_FC * _NSUB)):
        nb = sid * _NPT + j * _FC
        pltpu.sync_copy(ebuf.at[pl.ds(0, _FC)], S_sh.at[pl.ds(nb, _FC)])
        pltpu.sync_copy(ebuf.at[pl.ds(0, _FC)], W_sh.at[pl.ds(nb, _FC)])
    plsc.subcore_barrier()

    def process(off, csz, sv, dv):
        pltpu.sync_copy(src_hbm.at[pl.ds(off, csz)], sv)
        pltpu.sync_copy(dst_hbm.at[pl.ds(off, csz)], dv)
        pltpu.async_copy(h_hbm.at[cid].at[sv], hrows.at[pl.ds(0, csz)], sem).wait()
        pltpu.sync_copy(emb_hbm.at[cid, pl.ds(off, csz)], erows.at[pl.ds(0, csz)])

        def cbody(e, carry):
            for k in range(4):
                sl = pl.ds(k * 16, 16)
                msg = jnp.maximum(hrows[e, sl] + erows[e, sl], 0.0) + EPS
                ex = jnp.exp(msg)
                ebuf[e, sl] = ex
                wbuf[e, sl] = ex * msg
            return carry

        lax.fori_loop(0, csz, cbody, 0, unroll=False)
        pltpu.sync_copy(ebuf.at[pl.ds(0, csz)], S_sh.at[dv], add=True)
        pltpu.sync_copy(wbuf.at[pl.ds(0, csz)], W_sh.at[dv], add=True)

    base = sid * _EPT

    def chunk(i, carry):
        process(base + i * _C, _C, srcv, dstv)
        return carry

    lax.fori_loop(0, _NFULL, chunk, 0, unroll=False)
    if _CT:
        process(base + _NFULL * _C, _CT, srcvt, dstvt)

    plsc.subcore_barrier()

    for j in range(N // (_FC * _NSUB)):
        nb = sid * _NPT + j * _FC
        pltpu.sync_copy(S_sh.at[pl.ds(nb, _FC)], ebuf.at[pl.ds(0, _FC)])
        pltpu.sync_copy(W_sh.at[pl.ds(nb, _FC)], wbuf.at[pl.ds(0, _FC)])

        def fbody(e, carry):
            for k in range(4):
                sl = pl.ds(k * 16, 16)
                hrows[e, sl] = wbuf[e, sl] / (ebuf[e, sl] + 1e-16)
            return carry

        lax.fori_loop(0, _FC, fbody, 0, unroll=False)
        pltpu.sync_copy(hrows.at[pl.ds(0, _FC)], out_hbm.at[cid, pl.ds(nb, _FC)])


def _conv_sc(h_split, emb_split, src, dst):
    kern = pl.kernel(
        _conv_body,
        out_type=jax.ShapeDtypeStruct((2, N, H2), jnp.float32),
        mesh=_mesh,
        scratch_types=[
            pltpu.VMEM((_C,), jnp.int32),
            pltpu.VMEM((_C,), jnp.int32),
            pltpu.VMEM((_CT,), jnp.int32),
            pltpu.VMEM((_CT,), jnp.int32),
            pltpu.VMEM((_C, H2), jnp.float32),
            pltpu.VMEM((_C, H2), jnp.float32),
            pltpu.VMEM((_C, H2), jnp.float32),
            pltpu.VMEM((_C, H2), jnp.float32),
            pltpu.VMEM_SHARED((N, H2), jnp.float32),
            pltpu.VMEM_SHARED((N, H2), jnp.float32),
            pltpu.SemaphoreType.DMA,
        ],
        compiler_params=pltpu.CompilerParams(use_tc_tiling_on_sc=False),
    )
    return kern(h_split, emb_split, src, dst)


def kernel(x, edge_index, edge_attr, W_node, b_node, W_edge, b_edge,
           Wc0, bc0, Wc1, bc1, g0, be0, g1, be1):
    src = edge_index[0]
    dst = edge_index[1]
    b_node = b_node.reshape(1, H)
    b_edge = b_edge.reshape(1, H)
    bc0 = bc0.reshape(1, H)
    bc1 = bc1.reshape(1, H)
    g0 = g0.reshape(1, H)
    g1 = g1.reshape(1, H)
    be0 = be0.reshape(1, H)
    be1 = be1.reshape(1, H)

    h0 = _enc_node(x, W_node, b_node)
    emb = _enc_edge(edge_attr, W_edge, b_edge)
    a1 = _conv_sc(h0, emb, src, dst)
    h2 = _mlp(h0, a1, Wc0, bc0, g0, be0, relu_out=True)
    a2 = _conv_sc(h2, emb, src, dst)
    return _mlp(h2, a2, Wc1, bc1, g1, be1, relu_out=False)
